# baseline (speedup 1.0000x reference)
import jax
import jax.numpy as jnp
from jax.experimental import pallas as pl
from jax.experimental.pallas import tpu as pltpu

_BLOCK_ROWS = 256


def _fused_kernel(xcat_ref, wdiag_ref, l1_ref, l2_ref, bcat_ref,
                  ocat_ref, s1_ref, s2_ref):
    f_out = s1_ref.shape[1]

    @pl.when(pl.program_id(0) == 0)
    def _init():
        scat = jnp.dot(xcat_ref[...], wdiag_ref[...],
                       preferred_element_type=jnp.float32)
        s1_ref[...] = scat[:, :f_out]
        s2_ref[...] = scat[:, f_out:]

    p1 = jnp.dot(l1_ref[...], s1_ref[...], preferred_element_type=jnp.float32)
    p2 = jnp.dot(l2_ref[...], s2_ref[...], preferred_element_type=jnp.float32)
    ocat_ref[...] = jnp.concatenate([p1, p2], axis=1) + bcat_ref[...]


def kernel(node_input, hyperedge_input, node_lap, hyperedge_lap, weight, bias):
    n, f_in = node_input.shape
    m = hyperedge_input.shape[0]
    f_out = weight.shape[1]

    xcat = jnp.concatenate([node_input, hyperedge_input], axis=1)
    wz = jnp.zeros_like(weight)
    wdiag = jnp.concatenate(
        [jnp.concatenate([weight, wz], axis=1),
         jnp.concatenate([wz, weight], axis=1)], axis=0)
    bcat = jnp.concatenate([bias, bias]).reshape(1, 2 * f_out)

    blk = _BLOCK_ROWS
    ocat = pl.pallas_call(
        _fused_kernel,
        grid=(n // blk,),
        in_specs=[
            pl.BlockSpec((n, 2 * f_in), lambda i: (0, 0)),
            pl.BlockSpec((2 * f_in, 2 * f_out), lambda i: (0, 0)),
            pl.BlockSpec((blk, n), lambda i: (i, 0)),
            pl.BlockSpec((blk, m), lambda i: (i, 0)),
            pl.BlockSpec((1, 2 * f_out), lambda i: (0, 0)),
        ],
        out_specs=pl.BlockSpec((blk, 2 * f_out), lambda i: (i, 0)),
        out_shape=jax.ShapeDtypeStruct((n, 2 * f_out), jnp.float32),
        scratch_shapes=[
            pltpu.VMEM((n, f_out), jnp.float32),
            pltpu.VMEM((m, f_out), jnp.float32),
        ],
        compiler_params=pltpu.CompilerParams(
            dimension_semantics=("arbitrary",),
        ),
    )(xcat, wdiag, node_lap, hyperedge_lap, bcat)
    return ocat[:, :f_out], ocat[:, f_out:]


# speedup vs baseline: 1.2223x; 1.2223x over previous
import jax
import jax.numpy as jnp
from jax.experimental import pallas as pl
from jax.experimental.pallas import tpu as pltpu

_BLOCK_ROWS = 256


def _fused_kernel(x1t_ref, x2t_ref, w_ref, l1_ref, l2_ref, b_ref,
                  o1t_ref, o2t_ref, s1_ref, s2_ref):
    @pl.when(pl.program_id(0) == 0)
    def _init():
        w = w_ref[...]
        dn = (((0,), (0,)), ((), ()))
        s1_ref[...] = jax.lax.dot_general(
            x1t_ref[...], w, dn, preferred_element_type=jnp.float32)
        s2_ref[...] = jax.lax.dot_general(
            x2t_ref[...], w, dn, preferred_element_type=jnp.float32)

    b = b_ref[...]
    p1 = jnp.dot(l1_ref[...], s1_ref[...],
                 preferred_element_type=jnp.float32) + b
    p2 = jnp.dot(l2_ref[...], s2_ref[...],
                 preferred_element_type=jnp.float32) + b
    o1t_ref[...] = p1.T
    o2t_ref[...] = p2.T


def kernel(node_input, hyperedge_input, node_lap, hyperedge_lap, weight, bias):
    n, f_in = node_input.shape
    m = hyperedge_input.shape[0]
    f_out = weight.shape[1]

    x1t = node_input.T
    x2t = hyperedge_input.T
    bias2d = bias.reshape(1, f_out)
    blk = _BLOCK_ROWS
    o1t, o2t = pl.pallas_call(
        _fused_kernel,
        grid=(n // blk,),
        in_specs=[
            pl.BlockSpec((f_in, n), lambda i: (0, 0)),
            pl.BlockSpec((f_in, m), lambda i: (0, 0)),
            pl.BlockSpec((f_in, f_out), lambda i: (0, 0)),
            pl.BlockSpec((blk, n), lambda i: (i, 0)),
            pl.BlockSpec((blk, m), lambda i: (i, 0)),
            pl.BlockSpec((1, f_out), lambda i: (0, 0)),
        ],
        out_specs=(
            pl.BlockSpec((f_out, blk), lambda i: (0, i)),
            pl.BlockSpec((f_out, blk), lambda i: (0, i)),
        ),
        out_shape=(
            jax.ShapeDtypeStruct((f_out, n), jnp.float32),
            jax.ShapeDtypeStruct((f_out, m), jnp.float32),
        ),
        scratch_shapes=[
            pltpu.VMEM((n, f_out), jnp.float32),
            pltpu.VMEM((m, f_out), jnp.float32),
        ],
        compiler_params=pltpu.CompilerParams(
            dimension_semantics=("arbitrary",),
        ),
    )(x1t, x2t, weight, node_lap, hyperedge_lap, bias2d)
    return o1t.T, o2t.T
